# baseline (speedup 1.0000x reference)
import functools

import jax
import jax.numpy as jnp
from jax import lax
from jax.experimental import pallas as pl
from jax.experimental.pallas import tpu as pltpu
from jax.experimental.pallas import tpu_sc as plsc

N = 100000
C = 80
LOSS_WEIGHT = 1.0

_LOG1P = (
    0.00014154995003740067,
    0.9954269624606464,
    -0.4640715364410414,
    0.21640940804825148,
    -0.054862552015608815,
)


def _log1p_poly(u):
    acc = jnp.float32(_LOG1P[-1])
    for c in _LOG1P[-2::-1]:
        acc = acc * u + jnp.float32(c)
    return acc


def _neg_parts(x):
    u = jnp.exp(-jnp.abs(x))
    sp = jnp.maximum(x, 0.0) + _log1p_poly(u)
    t = 1.0 / (1.0 + u)
    s = jnp.where(x >= 0, t, u * t)
    return sp, s


DCHUNK = 1 << 19
DGRID = -(-(N * C) // DCHUNK)
DSUB = 8192
DITER = DCHUNK // DSUB


def _dense_body(x_ref, out_ref):
    flat0 = pl.program_id(0) * DCHUNK
    fiota = (jax.lax.broadcasted_iota(jnp.int32, (64, 128), 0) * 128
             + jax.lax.broadcasted_iota(jnp.int32, (64, 128), 1))

    def body(j, acc):
        x = x_ref[pl.ds(j * DSUB, DSUB)].reshape(64, 128)
        sp, s = _neg_parts(x)
        negv = sp * (s * s)
        valid = (flat0 + j * DSUB) + fiota < N * C
        return acc + jnp.where(valid, negv, 0.0)

    acc = jax.lax.fori_loop(
        0, DITER, body, jnp.zeros((64, 128), jnp.float32))
    bsum = (0.75 * jnp.sum(acc)).reshape(1, 1)

    @pl.when(pl.program_id(0) == 0)
    def _init():
        out_ref[...] = jnp.zeros((1, 1), jnp.float32)

    out_ref[...] += bsum


def _dense_sum(pred_flat):
    return pl.pallas_call(
        _dense_body,
        grid=(DGRID,),
        in_specs=[pl.BlockSpec((DCHUNK,), lambda i: (i,))],
        out_specs=pl.BlockSpec((1, 1), lambda i: (0, 0)),
        out_shape=jax.ShapeDtypeStruct((1, 1), jnp.float32),
    )(pred_flat)


NC_SC = 2
NS_SC = 16
NW = NC_SC * NS_SC
BW = 3200
TAIL = N - (NW - 1) * BW
GCH = 128
NGATH = BW // GCH
NVEC = BW // 16


def _sc_body(label_hbm, score_hbm, predflat_hbm, out_hbm,
             lab_v, sc_v, idx_v, gat_v, acc_v, sem):
    cid = lax.axis_index("c")
    sid = lax.axis_index("s")
    wid = sid * NC_SC + cid
    base = wid * BW

    @pl.when(wid < NW - 1)
    def _full_copy():
        pltpu.sync_copy(label_hbm.at[pl.ds(base, BW)], lab_v)
        pltpu.sync_copy(score_hbm.at[pl.ds(base, BW)], sc_v)

    @pl.when(wid == NW - 1)
    def _tail_copy():
        pltpu.sync_copy(label_hbm.at[pl.ds(base, TAIL)],
                        lab_v.at[pl.ds(0, TAIL)])
        pltpu.sync_copy(score_hbm.at[pl.ds(base, TAIL)],
                        sc_v.at[pl.ds(0, TAIL)])

    def idx_body(j, carry):
        lab = lab_v[pl.ds(j * 16, 16)]
        labc = jnp.minimum(jnp.maximum(lab, 0), C - 1)
        rows = base + j * 16 + lax.iota(jnp.int32, 16)
        idx = jnp.minimum(rows * C + labc, N * C - 1)
        idx_v[pl.ds(j * 16, 16)] = idx
        return carry

    lax.fori_loop(0, NVEC, idx_body, 0)

    handles = []
    for k in range(NGATH):
        handles.append(
            pltpu.async_copy(
                predflat_hbm.at[idx_v.at[pl.ds(k * GCH, GCH)]],
                gat_v.at[pl.ds(k * GCH, GCH)],
                sem,
            )
        )
    for h in handles:
        h.wait()

    def corr_body(j, acc):
        x = gat_v[pl.ds(j * 16, 16)]
        lab = lab_v[pl.ds(j * 16, 16)]
        sc = sc_v[pl.ds(j * 16, 16)]
        rows = base + j * 16 + lax.iota(jnp.int32, 16)
        pos = (rows < N) & (lab >= 0) & (lab < C)
        sc = jnp.where(pos, sc, 0.0)
        sp, s = _neg_parts(x)
        negp = 0.75 * sp * (s * s)
        w = jnp.abs(sc - x)
        pos_loss = (sp - x * sc) * (w * w)
        return acc + jnp.where(pos, pos_loss - negp, 0.0)

    acc = lax.fori_loop(0, NVEC, corr_body, jnp.zeros((16,), jnp.float32))
    acc_v[...] = acc
    pltpu.sync_copy(acc_v, out_hbm.at[wid])


@functools.cache
def _make_sc_corr():
    return functools.partial(
        pl.kernel,
        out_type=jax.ShapeDtypeStruct((NW, 16), jnp.float32),
        mesh=plsc.VectorSubcoreMesh(core_axis_name="c", subcore_axis_name="s"),
        scratch_types=[
            pltpu.VMEM((BW,), jnp.int32),
            pltpu.VMEM((BW,), jnp.float32),
            pltpu.VMEM((BW,), jnp.int32),
            pltpu.VMEM((BW,), jnp.float32),
            pltpu.VMEM((16,), jnp.float32),
            pltpu.SemaphoreType.DMA,
        ],
    )(_sc_body)


def kernel(pred, label, score):
    pred_flat = pred.reshape(N * C)
    corr = _make_sc_corr()(label, score, pred_flat)
    dense = _dense_sum(pred_flat)
    total = dense[0, 0] + jnp.sum(corr)
    return (total * (LOSS_WEIGHT / (N * C))).astype(jnp.float32)


# speedup vs baseline: 1.2235x; 1.2235x over previous
import jax
import jax.numpy as jnp
from jax.experimental import pallas as pl

N = 100000
C = 80
LOSS_WEIGHT = 1.0

_LOG1P = (
    0.00014154995003740067,
    0.9954269624606464,
    -0.4640715364410414,
    0.21640940804825148,
    -0.054862552015608815,
)

BLK = 2000
GRID = N // BLK
SUB = 80
NSUB = BLK // SUB


def _body(x_ref, lab_ref, sc_ref, out_ref):
    cols = jax.lax.broadcasted_iota(jnp.int32, (SUB, C), 1)

    acc = jnp.zeros((SUB, C), jnp.float32)
    for j in range(NSUB):
        x = x_ref[pl.ds(j * SUB, SUB), :]
        lab = lab_ref[pl.ds(j * SUB, SUB), :]
        sc = sc_ref[pl.ds(j * SUB, SUB), :]

        u = jnp.exp(-jnp.abs(x))
        p = jnp.float32(_LOG1P[-1])
        for c in _LOG1P[-2::-1]:
            p = p * u + jnp.float32(c)
        sp = jnp.maximum(x, 0.0) + p
        t = 1.0 / (1.0 + u)
        s = jnp.where(x >= 0, t, u * t)
        negv = (0.75 * sp) * (s * s)

        w = sc - x
        pos_loss = (sp - x * sc) * (w * w)
        onehot = cols == lab
        acc = acc + negv + jnp.where(onehot, pos_loss - negv, 0.0)

    bsum = jnp.sum(acc).reshape(1, 1)

    @pl.when(pl.program_id(0) == 0)
    def _init():
        out_ref[...] = jnp.zeros((1, 1), jnp.float32)

    out_ref[...] += bsum


def kernel(pred, label, score):
    out = pl.pallas_call(
        _body,
        grid=(GRID,),
        in_specs=[
            pl.BlockSpec((BLK, C), lambda i: (i, 0)),
            pl.BlockSpec((BLK, 1), lambda i: (i, 0)),
            pl.BlockSpec((BLK, 1), lambda i: (i, 0)),
        ],
        out_specs=pl.BlockSpec((1, 1), lambda i: (0, 0)),
        out_shape=jax.ShapeDtypeStruct((1, 1), jnp.float32),
    )(pred, label[:, None], score[:, None])
    return (out[0, 0] * (LOSS_WEIGHT / (N * C))).astype(jnp.float32)
